# baseline (speedup 1.0000x reference)
import functools

import jax
import jax.numpy as jnp
from jax import lax
from jax.experimental import pallas as pl
from jax.experimental.pallas import tpu as pltpu
from jax.experimental.pallas import tpu_sc as plsc

_NC = 2
_NS = 16
_NW = _NC * _NS
_K = 128
_G = 64
_EPS = 1e-5
_BAL_A = 39
_BAL_DEN = 100


def _make_agg(n_pad, d, chunks_a, chunks_b, max_chunks):
  rows_per_tile = n_pad // _NS
  mesh = plsc.VectorSubcoreMesh(core_axis_name="c", subcore_axis_name="s")

  @functools.partial(
      pl.kernel,
      out_type=jax.ShapeDtypeStruct((_NC * n_pad, d), jnp.float32),
      mesh=mesh,
      scratch_types=[
          pltpu.VMEM((_K,), jnp.int32),
          pltpu.VMEM((_K,), jnp.int32),
          pltpu.VMEM((_K, d), jnp.float32),
          pltpu.VMEM_SHARED((n_pad, d), jnp.float32),
          pltpu.SemaphoreType.DMA,
      ],
  )
  def agg(h_hbm, src_hbm, dst_hbm, zeros_hbm, out_hbm, src_v, dst_v, rows_v,
          acc, sem):
    c = lax.axis_index("c")
    s = lax.axis_index("s")
    wid = s * _NC + c
    r0 = s * rows_per_tile
    pltpu.sync_copy(zeros_hbm.at[pl.ds(r0, rows_per_tile)],
                    acc.at[pl.ds(r0, rows_per_tile)])
    plsc.subcore_barrier()

    def body(g, carry):
      off = (wid * max_chunks + g) * _K
      pltpu.sync_copy(src_hbm.at[pl.ds(off, _K)], src_v)
      pltpu.sync_copy(dst_hbm.at[pl.ds(off, _K)], dst_v)
      pltpu.async_copy(h_hbm.at[src_v], rows_v, sem).wait()
      pltpu.sync_copy(rows_v, acc.at[dst_v], add=True)
      return carry

    my_chunks = jnp.where(c == 0, chunks_a, chunks_b)
    lax.fori_loop(0, my_chunks, body, 0)
    plsc.subcore_barrier()
    pltpu.sync_copy(acc.at[pl.ds(r0, rows_per_tile)],
                    out_hbm.at[pl.ds(c * n_pad + r0, rows_per_tile)])

  return agg


def _rsqrt_precise(x):
  r = lax.rsqrt(x)
  r = r * (1.5 - 0.5 * x * r * r)
  r = r * (1.5 - 0.5 * x * r * r)
  return r


def _bn_relu(z, g, b):
  m = jnp.mean(z, axis=0, keepdims=True)
  zc = z - m
  v = jnp.mean(zc * zc, axis=0, keepdims=True)
  return jnp.maximum(zc * _rsqrt_precise(v + _EPS) * g + b, 0.0)


def _make_mlp(n, n_pad, d):
  def body(h_ref, p_ref, w1_ref, b1_ref, g1_ref, be1_ref, w2_ref, b2_ref,
           g2_ref, be2_ref, o_ref):
    z = h_ref[...] + p_ref[0:n, :] + p_ref[n_pad:n_pad + n, :]
    z = jnp.dot(z, w1_ref[...], preferred_element_type=jnp.float32, precision=lax.Precision.HIGHEST)
    z = _bn_relu(z + b1_ref[...], g1_ref[...], be1_ref[...])
    z = jnp.dot(z, w2_ref[...], preferred_element_type=jnp.float32, precision=lax.Precision.HIGHEST)
    o_ref[...] = _bn_relu(z + b2_ref[...], g2_ref[...], be2_ref[...])

  return pl.pallas_call(
      body, out_shape=jax.ShapeDtypeStruct((n, d), jnp.float32))


def _make_mlp_pool(n, n_pad, d, out_dim):
  def body(h_ref, p_ref, w1_ref, b1_ref, g1_ref, be1_ref, w2_ref, b2_ref,
           g2_ref, be2_ref, batch_ref, fcw_ref, fcb_ref, o_ref):
    z = h_ref[...] + p_ref[0:n, :] + p_ref[n_pad:n_pad + n, :]
    z = jnp.dot(z, w1_ref[...], preferred_element_type=jnp.float32, precision=lax.Precision.HIGHEST)
    z = _bn_relu(z + b1_ref[...], g1_ref[...], be1_ref[...])
    z = jnp.dot(z, w2_ref[...], preferred_element_type=jnp.float32, precision=lax.Precision.HIGHEST)
    h2 = _bn_relu(z + b2_ref[...], g2_ref[...], be2_ref[...])
    oht = (lax.broadcasted_iota(jnp.int32, (_G, n), 0)
           == batch_ref[...]).astype(jnp.float32)
    sums = jnp.dot(oht, h2, preferred_element_type=jnp.float32, precision=lax.Precision.HIGHEST)
    cnt = jnp.sum(oht, axis=1, keepdims=True)
    pooled = sums / jnp.maximum(cnt, 1.0)
    o_ref[...] = (jnp.dot(pooled, fcw_ref[...], preferred_element_type=jnp.float32,
                          precision=lax.Precision.HIGHEST) + fcb_ref[...])

  return pl.pallas_call(
      body, out_shape=jax.ShapeDtypeStruct((_G, out_dim), jnp.float32))


def kernel(x, edge_index, batch, W1, b1, g1, be1, W2, b2, g2, be2, fcW, fcb):
  n, d = x.shape
  e = edge_index.shape[1]
  num_layers = W1.shape[0]
  out_dim = fcW.shape[1]

  n_pad = (n + 1 + _NS * 8 - 1) // (_NS * 8) * (_NS * 8)

  total_chunks = (e + _K - 1) // _K
  chunks_a = -(-(total_chunks * _BAL_A) // (_BAL_DEN * _NS))
  rem = max(0, total_chunks - chunks_a * _NS)
  chunks_b = -(-rem // _NS)
  max_chunks = max(chunks_a, chunks_b)

  src_e = edge_index[0]
  dst_e = edge_index[1]
  cap = _NW * max_chunks * _K
  src_flat = jnp.zeros((cap,), jnp.int32)
  dst_flat = jnp.full((cap,), n, jnp.int32)
  epos = 0
  for w in range(_NW):
    cnt = (chunks_a if w % _NC == 0 else chunks_b) * _K
    take = min(cnt, e - epos)
    if take > 0:
      seg_s = lax.dynamic_slice(src_e, (epos,), (take,))
      seg_d = lax.dynamic_slice(dst_e, (epos,), (take,))
      src_flat = lax.dynamic_update_slice(src_flat, seg_s,
                                          (w * max_chunks * _K,))
      dst_flat = lax.dynamic_update_slice(dst_flat, seg_d,
                                          (w * max_chunks * _K,))
      epos += take
  zeros = jnp.zeros((n_pad, d), jnp.float32)

  agg_fn = _make_agg(n_pad, d, chunks_a, chunks_b, max_chunks)
  mlp_fn = _make_mlp(n, n_pad, d)
  mlp_pool_fn = _make_mlp_pool(n, n_pad, d, out_dim)

  r1 = lambda a: a.reshape(1, -1)
  h = x
  for l in range(num_layers):
    parts = agg_fn(h, src_flat, dst_flat, zeros)
    args = (h, parts, W1[l], r1(b1[l]), r1(g1[l]), r1(be1[l]),
            W2[l], r1(b2[l]), r1(g2[l]), r1(be2[l]))
    if l + 1 < num_layers:
      h = mlp_fn(*args)
    else:
      return mlp_pool_fn(*args, batch.reshape(1, n), fcW, r1(fcb))


# speedup vs baseline: 1.1443x; 1.1443x over previous
import functools

import jax
import jax.numpy as jnp
from jax import lax
from jax.experimental import pallas as pl
from jax.experimental.pallas import tpu as pltpu
from jax.experimental.pallas import tpu_sc as plsc

_NC = 2
_NS = 16
_NW = _NC * _NS
_K = 128
_G = 64
_EPS = 1e-5
_BAL_A = 61
_BAL_DEN = 100


def _make_agg(n_pad, d, chunks_a, chunks_b, max_chunks):
  rows_per_tile = n_pad // _NS
  mesh = plsc.VectorSubcoreMesh(core_axis_name="c", subcore_axis_name="s")

  @functools.partial(
      pl.kernel,
      out_type=jax.ShapeDtypeStruct((_NC * n_pad, d), jnp.float32),
      mesh=mesh,
      scratch_types=[
          pltpu.VMEM((_K,), jnp.int32),
          pltpu.VMEM((_K,), jnp.int32),
          pltpu.VMEM((_K, d), jnp.float32),
          pltpu.VMEM_SHARED((n_pad, d), jnp.float32),
          pltpu.SemaphoreType.DMA,
      ],
  )
  def agg(h_hbm, src_hbm, dst_hbm, zeros_hbm, out_hbm, src_v, dst_v, rows_v,
          acc, sem):
    c = lax.axis_index("c")
    s = lax.axis_index("s")
    wid = s * _NC + c
    r0 = s * rows_per_tile
    pltpu.sync_copy(zeros_hbm.at[pl.ds(r0, rows_per_tile)],
                    acc.at[pl.ds(r0, rows_per_tile)])
    plsc.subcore_barrier()

    def body(g, carry):
      off = (wid * max_chunks + g) * _K
      pltpu.sync_copy(src_hbm.at[pl.ds(off, _K)], src_v)
      pltpu.sync_copy(dst_hbm.at[pl.ds(off, _K)], dst_v)
      pltpu.async_copy(h_hbm.at[src_v], rows_v, sem).wait()
      pltpu.sync_copy(rows_v, acc.at[dst_v], add=True)
      return carry

    my_chunks = jnp.where(c == 0, chunks_a, chunks_b)
    lax.fori_loop(0, my_chunks, body, 0)
    plsc.subcore_barrier()
    pltpu.sync_copy(acc.at[pl.ds(r0, rows_per_tile)],
                    out_hbm.at[pl.ds(c * n_pad + r0, rows_per_tile)])

  return agg


def _rsqrt_precise(x):
  r = lax.rsqrt(x)
  r = r * (1.5 - 0.5 * x * r * r)
  r = r * (1.5 - 0.5 * x * r * r)
  return r


def _bn_relu(z, g, b):
  m = jnp.mean(z, axis=0, keepdims=True)
  zc = z - m
  v = jnp.mean(zc * zc, axis=0, keepdims=True)
  return jnp.maximum(zc * _rsqrt_precise(v + _EPS) * g + b, 0.0)


def _make_mlp(n, n_pad, d):
  def body(h_ref, p_ref, w1_ref, b1_ref, g1_ref, be1_ref, w2_ref, b2_ref,
           g2_ref, be2_ref, o_ref):
    z = h_ref[...] + p_ref[0:n, :] + p_ref[n_pad:n_pad + n, :]
    z = jnp.dot(z, w1_ref[...], preferred_element_type=jnp.float32, precision=lax.Precision.HIGHEST)
    z = _bn_relu(z + b1_ref[...], g1_ref[...], be1_ref[...])
    z = jnp.dot(z, w2_ref[...], preferred_element_type=jnp.float32, precision=lax.Precision.HIGHEST)
    o_ref[...] = _bn_relu(z + b2_ref[...], g2_ref[...], be2_ref[...])

  return pl.pallas_call(
      body, out_shape=jax.ShapeDtypeStruct((n, d), jnp.float32))


def _make_mlp_pool(n, n_pad, d, out_dim):
  def body(h_ref, p_ref, w1_ref, b1_ref, g1_ref, be1_ref, w2_ref, b2_ref,
           g2_ref, be2_ref, batch_ref, fcw_ref, fcb_ref, o_ref):
    z = h_ref[...] + p_ref[0:n, :] + p_ref[n_pad:n_pad + n, :]
    z = jnp.dot(z, w1_ref[...], preferred_element_type=jnp.float32, precision=lax.Precision.HIGHEST)
    z = _bn_relu(z + b1_ref[...], g1_ref[...], be1_ref[...])
    z = jnp.dot(z, w2_ref[...], preferred_element_type=jnp.float32, precision=lax.Precision.HIGHEST)
    h2 = _bn_relu(z + b2_ref[...], g2_ref[...], be2_ref[...])
    oht = (lax.broadcasted_iota(jnp.int32, (_G, n), 0)
           == batch_ref[...]).astype(jnp.float32)
    sums = jnp.dot(oht, h2, preferred_element_type=jnp.float32, precision=lax.Precision.HIGHEST)
    cnt = jnp.sum(oht, axis=1, keepdims=True)
    pooled = sums / jnp.maximum(cnt, 1.0)
    o_ref[...] = (jnp.dot(pooled, fcw_ref[...], preferred_element_type=jnp.float32,
                          precision=lax.Precision.HIGHEST) + fcb_ref[...])

  return pl.pallas_call(
      body, out_shape=jax.ShapeDtypeStruct((_G, out_dim), jnp.float32))


def kernel(x, edge_index, batch, W1, b1, g1, be1, W2, b2, g2, be2, fcW, fcb):
  n, d = x.shape
  e = edge_index.shape[1]
  num_layers = W1.shape[0]
  out_dim = fcW.shape[1]

  n_pad = (n + 1 + _NS * 8 - 1) // (_NS * 8) * (_NS * 8)

  total_chunks = (e + _K - 1) // _K
  chunks_a = -(-(total_chunks * _BAL_A) // (_BAL_DEN * _NS))
  rem = max(0, total_chunks - chunks_a * _NS)
  chunks_b = -(-rem // _NS)
  max_chunks = max(chunks_a, chunks_b)

  src_e = edge_index[0]
  dst_e = edge_index[1]
  cap = _NW * max_chunks * _K
  src_flat = jnp.zeros((cap,), jnp.int32)
  dst_flat = jnp.full((cap,), n, jnp.int32)
  epos = 0
  for w in range(_NW):
    cnt = (chunks_a if w % _NC == 0 else chunks_b) * _K
    take = min(cnt, e - epos)
    if take > 0:
      seg_s = lax.dynamic_slice(src_e, (epos,), (take,))
      seg_d = lax.dynamic_slice(dst_e, (epos,), (take,))
      src_flat = lax.dynamic_update_slice(src_flat, seg_s,
                                          (w * max_chunks * _K,))
      dst_flat = lax.dynamic_update_slice(dst_flat, seg_d,
                                          (w * max_chunks * _K,))
      epos += take
  zeros = jnp.zeros((n_pad, d), jnp.float32)

  agg_fn = _make_agg(n_pad, d, chunks_a, chunks_b, max_chunks)
  mlp_fn = _make_mlp(n, n_pad, d)
  mlp_pool_fn = _make_mlp_pool(n, n_pad, d, out_dim)

  r1 = lambda a: a.reshape(1, -1)
  h = x
  for l in range(num_layers):
    parts = agg_fn(h, src_flat, dst_flat, zeros)
    args = (h, parts, W1[l], r1(b1[l]), r1(g1[l]), r1(be1[l]),
            W2[l], r1(b2[l]), r1(g2[l]), r1(be2[l]))
    if l + 1 < num_layers:
      h = mlp_fn(*args)
    else:
      return mlp_pool_fn(*args, batch.reshape(1, n), fcW, r1(fcb))


# speedup vs baseline: 1.2533x; 1.0953x over previous
import functools

import jax
import jax.numpy as jnp
from jax import lax
from jax.experimental import pallas as pl
from jax.experimental.pallas import tpu as pltpu
from jax.experimental.pallas import tpu_sc as plsc

_NC = 2
_NS = 16
_NW = _NC * _NS
_K = 128
_G = 64
_EPS = 1e-5
_BAL_A = 1
_BAL_DEN = 2


def _make_agg(n_pad, d, chunks_a, chunks_b, max_chunks):
  rows_per_tile = n_pad // _NS
  mesh = plsc.VectorSubcoreMesh(core_axis_name="c", subcore_axis_name="s")

  @functools.partial(
      pl.kernel,
      out_type=jax.ShapeDtypeStruct((_NC * n_pad, d), jnp.float32),
      mesh=mesh,
      scratch_types=[
          pltpu.VMEM((_K,), jnp.int32),
          pltpu.VMEM((_K,), jnp.int32),
          pltpu.VMEM((_K, d), jnp.float32),
          pltpu.VMEM_SHARED((n_pad, d), jnp.float32),
          pltpu.SemaphoreType.DMA,
      ],
  )
  def agg(h_hbm, src_hbm, dst_hbm, zeros_hbm, out_hbm, src_v, dst_v, rows_v,
          acc, sem):
    c = lax.axis_index("c")
    s = lax.axis_index("s")
    wid = s * _NC + c
    r0 = s * rows_per_tile
    pltpu.sync_copy(zeros_hbm.at[pl.ds(r0, rows_per_tile)],
                    acc.at[pl.ds(r0, rows_per_tile)])
    plsc.subcore_barrier()

    def body(g, carry):
      off = (wid * max_chunks + g) * _K
      pltpu.sync_copy(src_hbm.at[pl.ds(off, _K)], src_v)
      pltpu.sync_copy(dst_hbm.at[pl.ds(off, _K)], dst_v)
      pltpu.async_copy(h_hbm.at[src_v], rows_v, sem).wait()
      pltpu.sync_copy(rows_v, acc.at[dst_v], add=True)
      return carry

    my_chunks = jnp.where(c == 0, chunks_a, chunks_b)
    lax.fori_loop(0, my_chunks, body, 0)
    plsc.subcore_barrier()
    pltpu.sync_copy(acc.at[pl.ds(r0, rows_per_tile)],
                    out_hbm.at[pl.ds(c * n_pad + r0, rows_per_tile)])

  return agg


def _rsqrt_precise(x):
  r = lax.rsqrt(x)
  r = r * (1.5 - 0.5 * x * r * r)
  r = r * (1.5 - 0.5 * x * r * r)
  return r


def _bn_relu(z, g, b):
  m = jnp.mean(z, axis=0, keepdims=True)
  zc = z - m
  v = jnp.mean(zc * zc, axis=0, keepdims=True)
  return jnp.maximum(zc * _rsqrt_precise(v + _EPS) * g + b, 0.0)


def _make_mlp(n, n_pad, d):
  def body(h_ref, p_ref, w1_ref, b1_ref, g1_ref, be1_ref, w2_ref, b2_ref,
           g2_ref, be2_ref, o_ref):
    z = h_ref[...] + p_ref[0:n, :] + p_ref[n_pad:n_pad + n, :]
    z = jnp.dot(z, w1_ref[...], preferred_element_type=jnp.float32, precision=lax.Precision.HIGHEST)
    z = _bn_relu(z + b1_ref[...], g1_ref[...], be1_ref[...])
    z = jnp.dot(z, w2_ref[...], preferred_element_type=jnp.float32, precision=lax.Precision.HIGHEST)
    o_ref[...] = _bn_relu(z + b2_ref[...], g2_ref[...], be2_ref[...])

  return pl.pallas_call(
      body, out_shape=jax.ShapeDtypeStruct((n, d), jnp.float32))


def _make_mlp_pool(n, n_pad, d, out_dim):
  def body(h_ref, p_ref, w1_ref, b1_ref, g1_ref, be1_ref, w2_ref, b2_ref,
           g2_ref, be2_ref, batch_ref, fcw_ref, fcb_ref, o_ref):
    z = h_ref[...] + p_ref[0:n, :] + p_ref[n_pad:n_pad + n, :]
    z = jnp.dot(z, w1_ref[...], preferred_element_type=jnp.float32, precision=lax.Precision.HIGHEST)
    z = _bn_relu(z + b1_ref[...], g1_ref[...], be1_ref[...])
    z = jnp.dot(z, w2_ref[...], preferred_element_type=jnp.float32, precision=lax.Precision.HIGHEST)
    h2 = _bn_relu(z + b2_ref[...], g2_ref[...], be2_ref[...])
    oht = (lax.broadcasted_iota(jnp.int32, (_G, n), 0)
           == batch_ref[...]).astype(jnp.float32)
    sums = jnp.dot(oht, h2, preferred_element_type=jnp.float32, precision=lax.Precision.HIGHEST)
    cnt = jnp.sum(oht, axis=1, keepdims=True)
    pooled = sums / jnp.maximum(cnt, 1.0)
    o_ref[...] = (jnp.dot(pooled, fcw_ref[...], preferred_element_type=jnp.float32,
                          precision=lax.Precision.HIGHEST) + fcb_ref[...])

  return pl.pallas_call(
      body, out_shape=jax.ShapeDtypeStruct((_G, out_dim), jnp.float32))


def kernel(x, edge_index, batch, W1, b1, g1, be1, W2, b2, g2, be2, fcW, fcb):
  n, d = x.shape
  e = edge_index.shape[1]
  num_layers = W1.shape[0]
  out_dim = fcW.shape[1]

  n_pad = (n + 1 + _NS * 8 - 1) // (_NS * 8) * (_NS * 8)

  total_chunks = (e + _K - 1) // _K
  chunks_a = -(-(total_chunks * _BAL_A) // (_BAL_DEN * _NS))
  rem = max(0, total_chunks - chunks_a * _NS)
  chunks_b = -(-rem // _NS)
  max_chunks = max(chunks_a, chunks_b)

  src_e = edge_index[0]
  dst_e = edge_index[1]
  cap = _NW * max_chunks * _K
  src_flat = jnp.zeros((cap,), jnp.int32)
  dst_flat = jnp.full((cap,), n, jnp.int32)
  epos = 0
  for w in range(_NW):
    cnt = (chunks_a if w % _NC == 0 else chunks_b) * _K
    take = min(cnt, e - epos)
    if take > 0:
      seg_s = lax.dynamic_slice(src_e, (epos,), (take,))
      seg_d = lax.dynamic_slice(dst_e, (epos,), (take,))
      src_flat = lax.dynamic_update_slice(src_flat, seg_s,
                                          (w * max_chunks * _K,))
      dst_flat = lax.dynamic_update_slice(dst_flat, seg_d,
                                          (w * max_chunks * _K,))
      epos += take
  zeros = jnp.zeros((n_pad, d), jnp.float32)

  agg_fn = _make_agg(n_pad, d, chunks_a, chunks_b, max_chunks)
  mlp_fn = _make_mlp(n, n_pad, d)
  mlp_pool_fn = _make_mlp_pool(n, n_pad, d, out_dim)

  r1 = lambda a: a.reshape(1, -1)
  h = x
  for l in range(num_layers):
    parts = agg_fn(h, src_flat, dst_flat, zeros)
    args = (h, parts, W1[l], r1(b1[l]), r1(g1[l]), r1(be1[l]),
            W2[l], r1(b2[l]), r1(g2[l]), r1(be2[l]))
    if l + 1 < num_layers:
      h = mlp_fn(*args)
    else:
      return mlp_pool_fn(*args, batch.reshape(1, n), fcW, r1(fcb))
